# Initial kernel scaffold; baseline (speedup 1.0000x reference)
import jax
import jax.numpy as jnp
from jax.experimental import pallas as pl


def kernel(x, emb, W, b):
    raise NotImplementedError("write your pallas kernel here")


# speedup vs baseline: 2.0309x; 2.0309x over previous
import functools

import jax
import jax.numpy as jnp
from jax import lax
from jax.experimental import pallas as pl
from jax.experimental.pallas import tpu as pltpu
from jax.experimental.pallas import tpu_sc as plsc

VOCAB = 1000000
EMB = 32
BATCH = 4096
L = 200

_info = plsc.get_sparse_core_info()
_NC, _NS = _info.num_cores, _info.num_subcores
NW = _NC * _NS
BAGS_PER_W = BATCH // NW
IDX_PER_W = BAGS_PER_W * L
C0 = 128
C1 = L - C0


def _sc_body(x_hbm, emb_hbm, w_hbm, out_hbm, idx_v, buf_v, w_v, part_v, out_v,
             sem):
    wid = lax.axis_index("s") * _NC + lax.axis_index("c")
    pltpu.sync_copy(x_hbm.at[wid], idx_v)
    pltpu.sync_copy(w_hbm, w_v)
    w0 = w_v[pl.ds(0, 16)]
    w1 = w_v[pl.ds(16, 16)]
    zero = jnp.zeros((16,), jnp.float32)
    lane = lax.iota(jnp.int32, 16)

    def group_body(g, carry):
        def bag_body(k, carry2):
            base = (g * 16 + k) * L
            cp0 = pltpu.async_copy(emb_hbm.at[idx_v.at[pl.ds(base, C0)]],
                                   buf_v.at[pl.ds(0, C0)], sem)
            cp1 = pltpu.async_copy(emb_hbm.at[idx_v.at[pl.ds(base + C0, C1)]],
                                   buf_v.at[pl.ds(C0, C1)], sem)
            cp0.wait()
            cp1.wait()

            def row_body(r, accs):
                a0, a1, b0, b1 = accs
                rr = r * 2
                a0 = a0 + buf_v[rr, pl.ds(0, 16)]
                a1 = a1 + buf_v[rr, pl.ds(16, 16)]
                b0 = b0 + buf_v[rr + 1, pl.ds(0, 16)]
                b1 = b1 + buf_v[rr + 1, pl.ds(16, 16)]
                return (a0, a1, b0, b1)

            a0, a1, b0, b1 = lax.fori_loop(0, L // 2, row_body,
                                           (zero, zero, zero, zero))
            part_v[k] = (a0 + b0) * w0 + (a1 + b1) * w1
            return carry2

        lax.fori_loop(0, 16, bag_body, 0)
        out_vec = zero
        for l in range(16):
            col = plsc.load_gather(part_v, [lane, jnp.full((16,), l, jnp.int32)])
            out_vec = out_vec + col
        out_v[pl.ds(g * 16, 16)] = out_vec
        return carry

    lax.fori_loop(0, BAGS_PER_W // 16, group_body, 0)
    pltpu.sync_copy(out_v, out_hbm.at[pl.ds(wid * BAGS_PER_W, BAGS_PER_W)])


@jax.jit
def _impl(x, emb, W, b):
    xr = x.astype(jnp.int32).reshape(NW, IDX_PER_W)
    wv = W.reshape(EMB)
    sc = pl.kernel(
        _sc_body,
        out_type=jax.ShapeDtypeStruct((BATCH,), jnp.float32),
        mesh=plsc.VectorSubcoreMesh(core_axis_name="c", subcore_axis_name="s"),
        compiler_params=pltpu.CompilerParams(needs_layout_passes=False,
                                             use_tc_tiling_on_sc=False),
        scratch_types=[
            pltpu.VMEM((IDX_PER_W,), jnp.int32),
            pltpu.VMEM((L, EMB), jnp.float32),
            pltpu.VMEM((EMB,), jnp.float32),
            pltpu.VMEM((16, 16), jnp.float32),
            pltpu.VMEM((BAGS_PER_W,), jnp.float32),
            pltpu.SemaphoreType.DMA,
        ],
    )
    dots = sc(xr, emb, wv)
    return dots.reshape(BATCH, 1) + b


def kernel(x, emb, W, b):
    return _impl(x, emb, W, b)


# speedup vs baseline: 2.3168x; 1.1408x over previous
import functools

import jax
import jax.numpy as jnp
from jax import lax
from jax.experimental import pallas as pl
from jax.experimental.pallas import tpu as pltpu
from jax.experimental.pallas import tpu_sc as plsc

VOCAB = 1000000
EMB = 32
BATCH = 4096
L = 200

_info = plsc.get_sparse_core_info()
_NC, _NS = _info.num_cores, _info.num_subcores
NW = _NC * _NS
BAGS_PER_W = BATCH // NW
IDX_PER_W = BAGS_PER_W * L
C0 = 128
C1 = L - C0


def _sc_body(x_hbm, emb_hbm, w_hbm, out_hbm, idx_v, buf0_v, buf1_v, w_v,
             part_v, out_v, sem0, sem1):
    wid = lax.axis_index("s") * _NC + lax.axis_index("c")
    pltpu.sync_copy(x_hbm.at[wid], idx_v)
    pltpu.sync_copy(w_hbm, w_v)
    w0 = w_v[pl.ds(0, 16)]
    w1 = w_v[pl.ds(16, 16)]
    zero = jnp.zeros((16,), jnp.float32)
    lane = lax.iota(jnp.int32, 16)

    def fire(buf, semx, t):
        base = t * L
        pltpu.async_copy(emb_hbm.at[idx_v.at[pl.ds(base, C0)]],
                         buf.at[pl.ds(0, C0)], semx)
        pltpu.async_copy(emb_hbm.at[idx_v.at[pl.ds(base + C0, C1)]],
                         buf.at[pl.ds(C0, C1)], semx)

    def wait_buf(buf, semx):
        pltpu.make_async_copy(emb_hbm.at[idx_v.at[pl.ds(0, C0)]],
                              buf.at[pl.ds(0, C0)], semx).wait()
        pltpu.make_async_copy(emb_hbm.at[idx_v.at[pl.ds(0, C1)]],
                              buf.at[pl.ds(C0, C1)], semx).wait()

    def consume(buf):
        def row_body(r, accs):
            a0, a1, b0, b1 = accs
            rr = r * 8
            for q in range(0, 8, 2):
                a0 = a0 + buf[rr + q, pl.ds(0, 16)]
                a1 = a1 + buf[rr + q, pl.ds(16, 16)]
                b0 = b0 + buf[rr + q + 1, pl.ds(0, 16)]
                b1 = b1 + buf[rr + q + 1, pl.ds(16, 16)]
            return (a0, a1, b0, b1)

        a0, a1, b0, b1 = lax.fori_loop(0, L // 8, row_body,
                                       (zero, zero, zero, zero))
        return (a0 + b0) * w0 + (a1 + b1) * w1

    fire(buf0_v, sem0, 0)

    def group_body(g, carry):
        def pair_body(p, carry2):
            t = g * 16 + p * 2
            fire(buf1_v, sem1, t + 1)
            wait_buf(buf0_v, sem0)
            part_v[p * 2] = consume(buf0_v)

            @pl.when(t + 2 < BAGS_PER_W)
            def _():
                fire(buf0_v, sem0, t + 2)

            wait_buf(buf1_v, sem1)
            part_v[p * 2 + 1] = consume(buf1_v)
            return carry2

        lax.fori_loop(0, 8, pair_body, 0)
        out_vec = zero
        for l in range(16):
            col = plsc.load_gather(part_v, [lane, jnp.full((16,), l, jnp.int32)])
            out_vec = out_vec + col
        out_v[pl.ds(g * 16, 16)] = out_vec
        return carry

    lax.fori_loop(0, BAGS_PER_W // 16, group_body, 0)
    pltpu.sync_copy(out_v, out_hbm.at[pl.ds(wid * BAGS_PER_W, BAGS_PER_W)])


@jax.jit
def _impl(x, emb, W, b):
    xr = x.astype(jnp.int32).reshape(NW, IDX_PER_W)
    wv = W.reshape(EMB)
    sc = pl.kernel(
        _sc_body,
        out_type=jax.ShapeDtypeStruct((BATCH,), jnp.float32),
        mesh=plsc.VectorSubcoreMesh(core_axis_name="c", subcore_axis_name="s"),
        compiler_params=pltpu.CompilerParams(needs_layout_passes=False,
                                             use_tc_tiling_on_sc=False),
        scratch_types=[
            pltpu.VMEM((IDX_PER_W,), jnp.int32),
            pltpu.VMEM((L, EMB), jnp.float32),
            pltpu.VMEM((L, EMB), jnp.float32),
            pltpu.VMEM((EMB,), jnp.float32),
            pltpu.VMEM((16, 16), jnp.float32),
            pltpu.VMEM((BAGS_PER_W,), jnp.float32),
            pltpu.SemaphoreType.DMA,
            pltpu.SemaphoreType.DMA,
        ],
    )
    dots = sc(xr, emb, wv)
    return dots.reshape(BATCH, 1) + b


def kernel(x, emb, W, b):
    return _impl(x, emb, W, b)
